# baseline (device time: 52063 ns/iter reference)
import jax
import jax.numpy as jnp
from jax import lax
from jax.experimental import pallas as pl
from jax.experimental.pallas import tpu as pltpu

N = 16
B = 2
SQ = 512
HL = 8
DH = 64
D_MODEL = 768
D_LOC = HL * DH
ROWS = B * SQ
NSCHED = 2
QR = SQ // 4
SR = QR // 4

SCHED_KINDS = [(0, 1), (1, 0)]
COLS = ((0, 512), (512, 256))


def _member(me, kind, idx):
    if kind == 0:
        return (me & ~3) + idx
    return (me & 3) + 4 * idx


class _BatchAllReduce:

    def __init__(self, b, me, acc_ref, agb_ref, rxq, sbs, rxs, sems):
        self.b = b
        self.me = me
        self.acc = acc_ref
        self.agb = agb_ref
        self.rxq = rxq
        self.sbs, self.rxs = sbs, rxs
        (self.rs0_s, self.rs0_r, self.rs1_s, self.rs1_r,
         self.ag0_s, self.ag0_r, self.ag1_s, self.ag1_r) = sems
        i = me & 3
        z = me >> 2
        self.g0 = [i if SCHED_KINDS[s][0] == 0 else z for s in range(NSCHED)]
        self.g1 = [z if SCHED_KINDS[s][0] == 0 else i for s in range(NSCHED)]
        self.q_off = [pl.multiple_of(b * SQ + self.g0[s] * QR, QR)
                      for s in range(NSCHED)]
        self.f_off = [pl.multiple_of(self.q_off[s] + self.g1[s] * SR, SR)
                      for s in range(NSCHED)]
        self.rdmas = [[None] * 3, [None] * 3]

    def _cols(self, s):
        return slice(COLS[s][0], COLS[s][0] + COLS[s][1])

    def start_rs0_half(self, half):
        for s in range(NSCHED):
            kind = SCHED_KINDS[s][0]
            for d in range(1, 4):
                tgt = (self.g0[s] + d) % 4
                off = pl.multiple_of(self.b * SQ + tgt * QR, QR)
                rdma = pltpu.make_async_remote_copy(
                    src_ref=self.agb.at[pl.ds(off, QR), self._cols(s)],
                    dst_ref=self.rxq[s].at[self.b, d - 1],
                    send_sem=self.rs0_s.at[self.b, s, d - 1],
                    recv_sem=self.rs0_r.at[self.b, s, d - 1],
                    device_id=(_member(self.me, kind, tgt),),
                    device_id_type=pl.DeviceIdType.MESH,
                )

                @pl.when(tgt // 2 == half)
                def _():
                    rdma.start()

                self.rdmas[s][d - 1] = rdma

    def finish_rs0(self):
        for s in range(NSCHED):
            for d in range(3):
                self.rdmas[s][d].wait()
            self.acc[pl.ds(self.q_off[s], QR), self._cols(s)] = (
                self.acc[pl.ds(self.q_off[s], QR), self._cols(s)]
                + self.rxq[s][self.b, 0].astype(jnp.float32)
                + self.rxq[s][self.b, 1].astype(jnp.float32)
                + self.rxq[s][self.b, 2].astype(jnp.float32))

    def start_rs1(self):
        for s in range(NSCHED):
            kind = SCHED_KINDS[s][1]
            for d in range(1, 4):
                tgt = (self.g1[s] + d) % 4
                off = pl.multiple_of(self.q_off[s] + tgt * SR, SR)
                self.sbs[s][self.b, d - 1, :, :] = self.acc[
                    pl.ds(off, SR), self._cols(s)].astype(jnp.bfloat16)
                rdma = pltpu.make_async_remote_copy(
                    src_ref=self.sbs[s].at[self.b, d - 1],
                    dst_ref=self.rxs[s].at[self.b, d - 1],
                    send_sem=self.rs1_s.at[self.b, s, d - 1],
                    recv_sem=self.rs1_r.at[self.b, s, d - 1],
                    device_id=(_member(self.me, kind, tgt),),
                    device_id_type=pl.DeviceIdType.MESH,
                )
                rdma.start()
                self.rdmas[s][d - 1] = rdma

    def finish_rs1(self):
        for s in range(NSCHED):
            for d in range(3):
                self.rdmas[s][d].wait()
            self.acc[pl.ds(self.f_off[s], SR), self._cols(s)] = (
                self.acc[pl.ds(self.f_off[s], SR), self._cols(s)]
                + self.rxs[s][self.b, 0].astype(jnp.float32)
                + self.rxs[s][self.b, 1].astype(jnp.float32)
                + self.rxs[s][self.b, 2].astype(jnp.float32))

    def start_ag0(self):
        for s in range(NSCHED):
            kind = SCHED_KINDS[s][1]
            self.agb[pl.ds(self.f_off[s], SR), self._cols(s)] = self.acc[
                pl.ds(self.f_off[s], SR), self._cols(s)].astype(jnp.bfloat16)
            for d in range(1, 4):
                tgt = (self.g1[s] + d) % 4
                rdma = pltpu.make_async_remote_copy(
                    src_ref=self.agb.at[pl.ds(self.f_off[s], SR),
                                        self._cols(s)],
                    dst_ref=self.agb.at[pl.ds(self.f_off[s], SR),
                                        self._cols(s)],
                    send_sem=self.ag0_s.at[self.b, s, d - 1],
                    recv_sem=self.ag0_r.at[self.b, s, d - 1],
                    device_id=(_member(self.me, kind, tgt),),
                    device_id_type=pl.DeviceIdType.MESH,
                )
                rdma.start()
                self.rdmas[s][d - 1] = rdma

    def finish_ag0(self):
        for s in range(NSCHED):
            for d in range(3):
                self.rdmas[s][d].wait()

    def start_ag1(self):
        for s in range(NSCHED):
            kind = SCHED_KINDS[s][0]
            for d in range(1, 4):
                tgt = (self.g0[s] + d) % 4
                rdma = pltpu.make_async_remote_copy(
                    src_ref=self.agb.at[pl.ds(self.q_off[s], QR),
                                        self._cols(s)],
                    dst_ref=self.agb.at[pl.ds(self.q_off[s], QR),
                                        self._cols(s)],
                    send_sem=self.ag1_s.at[self.b, s, d - 1],
                    recv_sem=self.ag1_r.at[self.b, s, d - 1],
                    device_id=(_member(self.me, kind, tgt),),
                    device_id_type=pl.DeviceIdType.MESH,
                )
                rdma.start()
                self.rdmas[s][d - 1] = rdma

    def finish_ag1(self):
        for s in range(NSCHED):
            for d in range(3):
                self.rdmas[s][d].wait()


def _compute_half(b, half, q, k_ref, v_ref, wo_ref, acc_ref, agb_ref, bias):
    HQ = SQ // 2
    r0 = half * HQ
    ke = HQ if half == 0 else SQ
    ctx_parts = []
    for h in range(HL):
        qh = q[r0:r0 + HQ, h * DH:(h + 1) * DH]
        kh = k_ref[b, :, h, :]
        vh = v_ref[b, :, h, :]
        s = jnp.dot(qh, kh[0:ke, :].T,
                    preferred_element_type=jnp.float32) * 0.125
        w = jnp.exp(s + bias[r0:r0 + HQ, 0:ke])
        ctx_parts.append(
            jnp.dot(w, vh[0:ke, :], preferred_element_type=jnp.float32)
            / jnp.sum(w, axis=-1, keepdims=True))
    ctx = jnp.concatenate(ctx_parts, axis=1)
    lo = b * SQ + r0
    acc_ref[lo:lo + HQ, :] = jnp.dot(
        ctx, wo_ref[:, :], preferred_element_type=jnp.float32)
    agb_ref[lo:lo + HQ, :] = acc_ref[lo:lo + HQ, :].astype(jnp.bfloat16)


def _body(x_ref, wq_ref, k_ref, v_ref, wo_ref, out_ref,
          acc_ref, agb_ref, rxq0, rxq1, sbs0, sbs1, rxs0, rxs1,
          rs0_s, rs0_r, rs1_s, rs1_r, ag0_s, ag0_r, ag1_s, ag1_r):
    me = lax.axis_index("i")
    rxq = [rxq0, rxq1]
    sbs, rxs = [sbs0, sbs1], [rxs0, rxs1]

    barrier = pltpu.get_barrier_semaphore()
    i = me & 3
    z = me >> 2
    for d in range(1, 4):
        pl.semaphore_signal(barrier, inc=1,
                            device_id=(_member(me, 0, (i + d) % 4),),
                            device_id_type=pl.DeviceIdType.MESH)
        pl.semaphore_signal(barrier, inc=1,
                            device_id=(_member(me, 1, (z + d) % 4),),
                            device_id_type=pl.DeviceIdType.MESH)
    pl.semaphore_wait(barrier, 6)

    qb = lax.broadcasted_iota(jnp.int32, (SQ, SQ), 0) // 64
    kb = lax.broadcasted_iota(jnp.int32, (SQ, SQ), 1) // 64
    bias = jnp.where(kb <= qb, 0.0, -1e9).astype(jnp.float32)

    sems = (rs0_s, rs0_r, rs1_s, rs1_r, ag0_s, ag0_r, ag1_s, ag1_r)
    ar0 = _BatchAllReduce(0, me, acc_ref, agb_ref, rxq, sbs, rxs, sems)
    ar1 = _BatchAllReduce(1, me, acc_ref, agb_ref, rxq, sbs, rxs, sems)

    q0 = jnp.dot(x_ref[0], wq_ref[:, :], preferred_element_type=jnp.float32)
    _compute_half(0, 0, q0, k_ref, v_ref, wo_ref, acc_ref, agb_ref, bias)
    ar0.start_rs0_half(0)
    _compute_half(0, 1, q0, k_ref, v_ref, wo_ref, acc_ref, agb_ref, bias)
    ar0.start_rs0_half(1)
    q1 = jnp.dot(x_ref[1], wq_ref[:, :], preferred_element_type=jnp.float32)
    _compute_half(1, 0, q1, k_ref, v_ref, wo_ref, acc_ref, agb_ref, bias)
    ar1.start_rs0_half(0)
    _compute_half(1, 1, q1, k_ref, v_ref, wo_ref, acc_ref, agb_ref, bias)
    ar1.start_rs0_half(1)

    ar0.finish_rs0(); ar0.start_rs1()
    ar0.finish_rs1(); ar0.start_ag0()
    ar1.finish_rs0(); ar1.start_rs1()
    ar0.finish_ag0(); ar0.start_ag1()
    ar1.finish_rs1(); ar1.start_ag0()
    ar0.finish_ag1()
    out_ref[0] = agb_ref[0:SQ, :].astype(jnp.float32)
    ar1.finish_ag0(); ar1.start_ag1()
    ar1.finish_ag1()
    out_ref[1] = agb_ref[SQ:ROWS, :].astype(jnp.float32)


def kernel(x, Wq, K_ext, V_ext, Wo):
    me = lax.axis_index("i")
    wq_loc = lax.dynamic_slice(Wq, (0, me * D_LOC), (Wq.shape[0], D_LOC))
    wo_loc = lax.dynamic_slice(Wo, (me * D_LOC, 0), (D_LOC, Wo.shape[1]))

    return pl.pallas_call(
        _body,
        out_shape=jax.ShapeDtypeStruct((B, SQ, D_MODEL), jnp.float32),
        in_specs=[pl.BlockSpec(memory_space=pltpu.VMEM)] * 5,
        out_specs=pl.BlockSpec(memory_space=pltpu.VMEM),
        scratch_shapes=[
            pltpu.VMEM((ROWS, D_MODEL), jnp.float32),
            pltpu.VMEM((ROWS, D_MODEL), jnp.bfloat16),
            pltpu.VMEM((B, 3, QR, COLS[0][1]), jnp.bfloat16),
            pltpu.VMEM((B, 3, QR, COLS[1][1]), jnp.bfloat16),
            pltpu.VMEM((B, 3, SR, COLS[0][1]), jnp.bfloat16),
            pltpu.VMEM((B, 3, SR, COLS[1][1]), jnp.bfloat16),
            pltpu.VMEM((B, 3, SR, COLS[0][1]), jnp.bfloat16),
            pltpu.VMEM((B, 3, SR, COLS[1][1]), jnp.bfloat16),
            pltpu.SemaphoreType.DMA((B, NSCHED, 3)),
            pltpu.SemaphoreType.DMA((B, NSCHED, 3)),
            pltpu.SemaphoreType.DMA((B, NSCHED, 3)),
            pltpu.SemaphoreType.DMA((B, NSCHED, 3)),
            pltpu.SemaphoreType.DMA((B, NSCHED, 3)),
            pltpu.SemaphoreType.DMA((B, NSCHED, 3)),
            pltpu.SemaphoreType.DMA((B, NSCHED, 3)),
            pltpu.SemaphoreType.DMA((B, NSCHED, 3)),
        ],
        compiler_params=pltpu.CompilerParams(collective_id=0),
    )(x, wq_loc, K_ext, V_ext, wo_loc)


# device time: 45627 ns/iter; 1.1411x vs baseline; 1.1411x over previous
import jax
import jax.numpy as jnp
from jax import lax
from jax.experimental import pallas as pl
from jax.experimental.pallas import tpu as pltpu

N = 16
B = 2
SQ = 512
HL = 8
DH = 64
D_MODEL = 768
D_LOC = HL * DH
ROWS = B * SQ
NSCHED = 2
QR = SQ // 4
SR = QR // 4

SCHED_KINDS = [(0, 1), (1, 0)]
COLS = ((0, 512), (512, 256))


def _member(me, kind, idx):
    if kind == 0:
        return (me & ~3) + idx
    return (me & 3) + 4 * idx


class _BatchAllReduce:

    def __init__(self, b, me, acc_ref, agb_ref, rxq, sbs, rxs, sems):
        self.b = b
        self.me = me
        self.acc = acc_ref
        self.agb = agb_ref
        self.rxq = rxq
        self.sbs, self.rxs = sbs, rxs
        (self.rs0_s, self.rs0_r, self.rs1_s, self.rs1_r,
         self.ag0_s, self.ag0_r, self.ag1_s, self.ag1_r) = sems
        i = me & 3
        z = me >> 2
        self.g0 = [i if SCHED_KINDS[s][0] == 0 else z for s in range(NSCHED)]
        self.g1 = [z if SCHED_KINDS[s][0] == 0 else i for s in range(NSCHED)]
        self.q_off = [pl.multiple_of(b * SQ + self.g0[s] * QR, QR)
                      for s in range(NSCHED)]
        self.f_off = [pl.multiple_of(self.q_off[s] + self.g1[s] * SR, SR)
                      for s in range(NSCHED)]
        self.rdmas = [[None] * 3, [None] * 3]

    def _cols(self, s):
        return slice(COLS[s][0], COLS[s][0] + COLS[s][1])

    def start_rs0(self):
        self.agb[self.b * SQ:(self.b + 1) * SQ, :] = self.acc[
            self.b * SQ:(self.b + 1) * SQ, :].astype(jnp.bfloat16)
        for s in range(NSCHED):
            kind = SCHED_KINDS[s][0]
            for d in range(1, 4):
                tgt = (self.g0[s] + d) % 4
                off = pl.multiple_of(self.b * SQ + tgt * QR, QR)
                rdma = pltpu.make_async_remote_copy(
                    src_ref=self.agb.at[pl.ds(off, QR), self._cols(s)],
                    dst_ref=self.rxq[s].at[self.b, d - 1],
                    send_sem=self.rs0_s.at[self.b, s, d - 1],
                    recv_sem=self.rs0_r.at[self.b, s, d - 1],
                    device_id=(_member(self.me, kind, tgt),),
                    device_id_type=pl.DeviceIdType.MESH,
                )
                rdma.start()
                self.rdmas[s][d - 1] = rdma

    def finish_rs0(self):
        for s in range(NSCHED):
            for d in range(3):
                self.rdmas[s][d].wait()
            self.acc[pl.ds(self.q_off[s], QR), self._cols(s)] = (
                self.acc[pl.ds(self.q_off[s], QR), self._cols(s)]
                + self.rxq[s][self.b, 0].astype(jnp.float32)
                + self.rxq[s][self.b, 1].astype(jnp.float32)
                + self.rxq[s][self.b, 2].astype(jnp.float32))

    def start_rs1(self):
        for s in range(NSCHED):
            kind = SCHED_KINDS[s][1]
            for d in range(1, 4):
                tgt = (self.g1[s] + d) % 4
                off = pl.multiple_of(self.q_off[s] + tgt * SR, SR)
                self.sbs[s][self.b, d - 1, :, :] = self.acc[
                    pl.ds(off, SR), self._cols(s)].astype(jnp.bfloat16)
                rdma = pltpu.make_async_remote_copy(
                    src_ref=self.sbs[s].at[self.b, d - 1],
                    dst_ref=self.rxs[s].at[self.b, d - 1],
                    send_sem=self.rs1_s.at[self.b, s, d - 1],
                    recv_sem=self.rs1_r.at[self.b, s, d - 1],
                    device_id=(_member(self.me, kind, tgt),),
                    device_id_type=pl.DeviceIdType.MESH,
                )
                rdma.start()
                self.rdmas[s][d - 1] = rdma

    def finish_rs1(self):
        for s in range(NSCHED):
            for d in range(3):
                self.rdmas[s][d].wait()
            self.acc[pl.ds(self.f_off[s], SR), self._cols(s)] = (
                self.acc[pl.ds(self.f_off[s], SR), self._cols(s)]
                + self.rxs[s][self.b, 0].astype(jnp.float32)
                + self.rxs[s][self.b, 1].astype(jnp.float32)
                + self.rxs[s][self.b, 2].astype(jnp.float32))

    def start_ag0(self):
        for s in range(NSCHED):
            kind = SCHED_KINDS[s][1]
            self.agb[pl.ds(self.f_off[s], SR), self._cols(s)] = self.acc[
                pl.ds(self.f_off[s], SR), self._cols(s)].astype(jnp.bfloat16)
            for d in range(1, 4):
                tgt = (self.g1[s] + d) % 4
                rdma = pltpu.make_async_remote_copy(
                    src_ref=self.agb.at[pl.ds(self.f_off[s], SR),
                                        self._cols(s)],
                    dst_ref=self.agb.at[pl.ds(self.f_off[s], SR),
                                        self._cols(s)],
                    send_sem=self.ag0_s.at[self.b, s, d - 1],
                    recv_sem=self.ag0_r.at[self.b, s, d - 1],
                    device_id=(_member(self.me, kind, tgt),),
                    device_id_type=pl.DeviceIdType.MESH,
                )
                rdma.start()
                self.rdmas[s][d - 1] = rdma

    def finish_ag0(self):
        for s in range(NSCHED):
            for d in range(3):
                self.rdmas[s][d].wait()

    def start_ag1(self):
        for s in range(NSCHED):
            kind = SCHED_KINDS[s][0]
            for d in range(1, 4):
                tgt = (self.g0[s] + d) % 4
                rdma = pltpu.make_async_remote_copy(
                    src_ref=self.agb.at[pl.ds(self.q_off[s], QR),
                                        self._cols(s)],
                    dst_ref=self.agb.at[pl.ds(self.q_off[s], QR),
                                        self._cols(s)],
                    send_sem=self.ag1_s.at[self.b, s, d - 1],
                    recv_sem=self.ag1_r.at[self.b, s, d - 1],
                    device_id=(_member(self.me, kind, tgt),),
                    device_id_type=pl.DeviceIdType.MESH,
                )
                rdma.start()
                self.rdmas[s][d - 1] = rdma

    def finish_ag1(self):
        for s in range(NSCHED):
            for d in range(3):
                self.rdmas[s][d].wait()


def _compute_batch(b, x_ref, wq_ref, k_ref, v_ref, wo_ref, acc_ref, bias):
    q = jnp.dot(x_ref[b], wq_ref[:, :],
                preferred_element_type=jnp.float32)
    ctx_parts = []
    HQ = SQ // 2
    for h in range(HL):
        qh = q[:, h * DH:(h + 1) * DH]
        kh = k_ref[b, :, h, :]
        vh = v_ref[b, :, h, :]
        s_lo = jnp.dot(qh[0:HQ, :], kh[0:HQ, :].T,
                       preferred_element_type=jnp.float32) * 0.125
        w_lo = jnp.exp(s_lo + bias[0:HQ, 0:HQ])
        ctx_lo = (jnp.dot(w_lo, vh[0:HQ, :],
                          preferred_element_type=jnp.float32)
                  / jnp.sum(w_lo, axis=-1, keepdims=True))
        s_hi = jnp.dot(qh[HQ:SQ, :], kh.T,
                       preferred_element_type=jnp.float32) * 0.125
        w_hi = jnp.exp(s_hi + bias[HQ:SQ, :])
        ctx_hi = (jnp.dot(w_hi, vh,
                          preferred_element_type=jnp.float32)
                  / jnp.sum(w_hi, axis=-1, keepdims=True))
        ctx_parts.append(jnp.concatenate([ctx_lo, ctx_hi], axis=0))
    ctx = jnp.concatenate(ctx_parts, axis=1)
    acc_ref[b * SQ:(b + 1) * SQ, :] = jnp.dot(
        ctx, wo_ref[:, :], preferred_element_type=jnp.float32)


def _body(x_ref, wq_ref, k_ref, v_ref, wo_ref, out_ref,
          acc_ref, agb_ref, rxq0, rxq1, sbs0, sbs1, rxs0, rxs1,
          rs0_s, rs0_r, rs1_s, rs1_r, ag0_s, ag0_r, ag1_s, ag1_r):
    me = lax.axis_index("i")
    rxq = [rxq0, rxq1]
    sbs, rxs = [sbs0, sbs1], [rxs0, rxs1]

    barrier = pltpu.get_barrier_semaphore()
    i = me & 3
    z = me >> 2
    for d in range(1, 4):
        pl.semaphore_signal(barrier, inc=1,
                            device_id=(_member(me, 0, (i + d) % 4),),
                            device_id_type=pl.DeviceIdType.MESH)
        pl.semaphore_signal(barrier, inc=1,
                            device_id=(_member(me, 1, (z + d) % 4),),
                            device_id_type=pl.DeviceIdType.MESH)
    pl.semaphore_wait(barrier, 6)

    qb = lax.broadcasted_iota(jnp.int32, (SQ, SQ), 0) // 64
    kb = lax.broadcasted_iota(jnp.int32, (SQ, SQ), 1) // 64
    bias = jnp.where(kb <= qb, 0.0, -1e9).astype(jnp.float32)

    sems = (rs0_s, rs0_r, rs1_s, rs1_r, ag0_s, ag0_r, ag1_s, ag1_r)
    ar0 = _BatchAllReduce(0, me, acc_ref, agb_ref, rxq, sbs, rxs, sems)
    ar1 = _BatchAllReduce(1, me, acc_ref, agb_ref, rxq, sbs, rxs, sems)

    _compute_batch(0, x_ref, wq_ref, k_ref, v_ref, wo_ref, acc_ref, bias)
    ar0.start_rs0()
    _compute_batch(1, x_ref, wq_ref, k_ref, v_ref, wo_ref, acc_ref, bias)

    ar1.start_rs0()
    ar0.finish_rs0(); ar0.start_rs1()
    ar0.finish_rs1(); ar0.start_ag0()
    ar1.finish_rs0(); ar1.start_rs1()
    ar0.finish_ag0(); ar0.start_ag1()
    ar1.finish_rs1(); ar1.start_ag0()
    ar0.finish_ag1()
    out_ref[0] = agb_ref[0:SQ, :].astype(jnp.float32)
    ar1.finish_ag0(); ar1.start_ag1()
    ar1.finish_ag1()
    out_ref[1] = agb_ref[SQ:ROWS, :].astype(jnp.float32)


def kernel(x, Wq, K_ext, V_ext, Wo):
    me = lax.axis_index("i")
    wq_loc = lax.dynamic_slice(Wq, (0, me * D_LOC), (Wq.shape[0], D_LOC))
    wo_loc = lax.dynamic_slice(Wo, (me * D_LOC, 0), (D_LOC, Wo.shape[1]))

    return pl.pallas_call(
        _body,
        out_shape=jax.ShapeDtypeStruct((B, SQ, D_MODEL), jnp.float32),
        in_specs=[pl.BlockSpec(memory_space=pltpu.VMEM)] * 5,
        out_specs=pl.BlockSpec(memory_space=pltpu.VMEM),
        scratch_shapes=[
            pltpu.VMEM((ROWS, D_MODEL), jnp.float32),
            pltpu.VMEM((ROWS, D_MODEL), jnp.bfloat16),
            pltpu.VMEM((B, 3, QR, COLS[0][1]), jnp.bfloat16),
            pltpu.VMEM((B, 3, QR, COLS[1][1]), jnp.bfloat16),
            pltpu.VMEM((B, 3, SR, COLS[0][1]), jnp.bfloat16),
            pltpu.VMEM((B, 3, SR, COLS[1][1]), jnp.bfloat16),
            pltpu.VMEM((B, 3, SR, COLS[0][1]), jnp.bfloat16),
            pltpu.VMEM((B, 3, SR, COLS[1][1]), jnp.bfloat16),
            pltpu.SemaphoreType.DMA((B, NSCHED, 3)),
            pltpu.SemaphoreType.DMA((B, NSCHED, 3)),
            pltpu.SemaphoreType.DMA((B, NSCHED, 3)),
            pltpu.SemaphoreType.DMA((B, NSCHED, 3)),
            pltpu.SemaphoreType.DMA((B, NSCHED, 3)),
            pltpu.SemaphoreType.DMA((B, NSCHED, 3)),
            pltpu.SemaphoreType.DMA((B, NSCHED, 3)),
            pltpu.SemaphoreType.DMA((B, NSCHED, 3)),
        ],
        compiler_params=pltpu.CompilerParams(collective_id=0),
    )(x, wq_loc, K_ext, V_ext, wo_loc)


# device time: 44209 ns/iter; 1.1777x vs baseline; 1.0321x over previous
import jax
import jax.numpy as jnp
from jax import lax
from jax.experimental import pallas as pl
from jax.experimental.pallas import tpu as pltpu

N = 16
B = 2
SQ = 512
HL = 8
DH = 64
D_MODEL = 768
D_LOC = HL * DH
ROWS = B * SQ
NSCHED = 2
QR = SQ // 4
SR = QR // 4

SCHED_KINDS = [(0, 1), (1, 0)]
COLS = ((0, 512), (512, 256))


def _member(me, kind, idx):
    if kind == 0:
        return (me & ~3) + idx
    return (me & 3) + 4 * idx


class _BatchAllReduce:

    def __init__(self, b, me, acc_ref, agb_ref, rxq, sbs, rxs, sems):
        self.b = b
        self.me = me
        self.acc = acc_ref
        self.agb = agb_ref
        self.rxq = rxq
        self.sbs, self.rxs = sbs, rxs
        (self.rs0_s, self.rs0_r, self.rs1_s, self.rs1_r,
         self.ag0_s, self.ag0_r, self.ag1_s, self.ag1_r) = sems
        i = me & 3
        z = me >> 2
        self.g0 = [i if SCHED_KINDS[s][0] == 0 else z for s in range(NSCHED)]
        self.g1 = [z if SCHED_KINDS[s][0] == 0 else i for s in range(NSCHED)]
        self.q_off = [pl.multiple_of(b * SQ + self.g0[s] * QR, QR)
                      for s in range(NSCHED)]
        self.f_off = [pl.multiple_of(self.q_off[s] + self.g1[s] * SR, SR)
                      for s in range(NSCHED)]
        self.rdmas = [[None] * 3, [None] * 3]

    def _cols(self, s):
        return slice(COLS[s][0], COLS[s][0] + COLS[s][1])

    def start_rs0(self):
        self.agb[self.b * SQ:(self.b + 1) * SQ, :] = self.acc[
            self.b * SQ:(self.b + 1) * SQ, :].astype(jnp.bfloat16)
        for s in range(NSCHED):
            kind = SCHED_KINDS[s][0]
            for d in range(1, 4):
                tgt = (self.g0[s] + d) % 4
                off = pl.multiple_of(self.b * SQ + tgt * QR, QR)
                rdma = pltpu.make_async_remote_copy(
                    src_ref=self.agb.at[pl.ds(off, QR), self._cols(s)],
                    dst_ref=self.rxq[s].at[self.b, d - 1],
                    send_sem=self.rs0_s.at[self.b, s, d - 1],
                    recv_sem=self.rs0_r.at[self.b, s, d - 1],
                    device_id=(_member(self.me, kind, tgt),),
                    device_id_type=pl.DeviceIdType.MESH,
                )
                rdma.start()
                self.rdmas[s][d - 1] = rdma

    def finish_rs0(self):
        for s in range(NSCHED):
            for d in range(3):
                self.rdmas[s][d].wait()
            self.acc[pl.ds(self.q_off[s], QR), self._cols(s)] = (
                self.acc[pl.ds(self.q_off[s], QR), self._cols(s)]
                + self.rxq[s][self.b, 0].astype(jnp.float32)
                + self.rxq[s][self.b, 1].astype(jnp.float32)
                + self.rxq[s][self.b, 2].astype(jnp.float32))

    def start_rs1(self):
        for s in range(NSCHED):
            kind = SCHED_KINDS[s][1]
            for d in range(1, 4):
                tgt = (self.g1[s] + d) % 4
                off = pl.multiple_of(self.q_off[s] + tgt * SR, SR)
                self.sbs[s][self.b, d - 1, :, :] = self.acc[
                    pl.ds(off, SR), self._cols(s)].astype(jnp.bfloat16)
                rdma = pltpu.make_async_remote_copy(
                    src_ref=self.sbs[s].at[self.b, d - 1],
                    dst_ref=self.rxs[s].at[self.b, d - 1],
                    send_sem=self.rs1_s.at[self.b, s, d - 1],
                    recv_sem=self.rs1_r.at[self.b, s, d - 1],
                    device_id=(_member(self.me, kind, tgt),),
                    device_id_type=pl.DeviceIdType.MESH,
                )
                rdma.start()
                self.rdmas[s][d - 1] = rdma

    def finish_rs1(self):
        for s in range(NSCHED):
            for d in range(3):
                self.rdmas[s][d].wait()
            self.acc[pl.ds(self.f_off[s], SR), self._cols(s)] = (
                self.acc[pl.ds(self.f_off[s], SR), self._cols(s)]
                + self.rxs[s][self.b, 0].astype(jnp.float32)
                + self.rxs[s][self.b, 1].astype(jnp.float32)
                + self.rxs[s][self.b, 2].astype(jnp.float32))

    def start_ag0(self):
        for s in range(NSCHED):
            kind = SCHED_KINDS[s][1]
            self.agb[pl.ds(self.f_off[s], SR), self._cols(s)] = self.acc[
                pl.ds(self.f_off[s], SR), self._cols(s)].astype(jnp.bfloat16)
            for d in range(1, 4):
                tgt = (self.g1[s] + d) % 4
                rdma = pltpu.make_async_remote_copy(
                    src_ref=self.agb.at[pl.ds(self.f_off[s], SR),
                                        self._cols(s)],
                    dst_ref=self.agb.at[pl.ds(self.f_off[s], SR),
                                        self._cols(s)],
                    send_sem=self.ag0_s.at[self.b, s, d - 1],
                    recv_sem=self.ag0_r.at[self.b, s, d - 1],
                    device_id=(_member(self.me, kind, tgt),),
                    device_id_type=pl.DeviceIdType.MESH,
                )
                rdma.start()
                self.rdmas[s][d - 1] = rdma

    def finish_ag0(self):
        for s in range(NSCHED):
            for d in range(3):
                self.rdmas[s][d].wait()

    def start_ag1(self):
        for s in range(NSCHED):
            kind = SCHED_KINDS[s][0]
            for d in range(1, 4):
                tgt = (self.g0[s] + d) % 4
                rdma = pltpu.make_async_remote_copy(
                    src_ref=self.agb.at[pl.ds(self.q_off[s], QR),
                                        self._cols(s)],
                    dst_ref=self.agb.at[pl.ds(self.q_off[s], QR),
                                        self._cols(s)],
                    send_sem=self.ag1_s.at[self.b, s, d - 1],
                    recv_sem=self.ag1_r.at[self.b, s, d - 1],
                    device_id=(_member(self.me, kind, tgt),),
                    device_id_type=pl.DeviceIdType.MESH,
                )
                rdma.start()
                self.rdmas[s][d - 1] = rdma

    def finish_ag1(self):
        for s in range(NSCHED):
            for d in range(3):
                self.rdmas[s][d].wait()


def _compute_batch(b, x_ref, wq_ref, k_ref, v_ref, wo_ref, acc_ref, bias):
    q = jnp.dot(x_ref[b], wq_ref[:, :],
                preferred_element_type=jnp.float32)
    ctx_parts = []
    HQ = SQ // 2
    for h in range(HL):
        qh = q[:, h * DH:(h + 1) * DH]
        kh = k_ref[b, :, h, :]
        vh = v_ref[b, :, h, :]
        s_lo = jnp.dot(qh[0:HQ, :], kh[0:HQ, :].T,
                       preferred_element_type=jnp.float32) * 0.125
        w_lo = jnp.exp(s_lo + bias[0:HQ, 0:HQ])
        ctx_lo = (jnp.dot(w_lo, vh[0:HQ, :],
                          preferred_element_type=jnp.float32)
                  / jnp.sum(w_lo, axis=-1, keepdims=True))
        s_hi = jnp.dot(qh[HQ:SQ, :], kh.T,
                       preferred_element_type=jnp.float32) * 0.125
        w_hi = jnp.exp(s_hi + bias[HQ:SQ, :])
        ctx_hi = (jnp.dot(w_hi, vh,
                          preferred_element_type=jnp.float32)
                  / jnp.sum(w_hi, axis=-1, keepdims=True))
        ctx_parts.append(jnp.concatenate([ctx_lo, ctx_hi], axis=0))
    ctx = jnp.concatenate(ctx_parts, axis=1)
    acc_ref[b * SQ:(b + 1) * SQ, :] = jnp.dot(
        ctx, wo_ref[:, :], preferred_element_type=jnp.float32)


def _body(x_ref, wq_ref, k_ref, v_ref, wo_ref, out_ref,
          acc_ref, agb_ref, rxq0, rxq1, sbs0, sbs1, rxs0, rxs1,
          rs0_s, rs0_r, rs1_s, rs1_r, ag0_s, ag0_r, ag1_s, ag1_r):
    me = lax.axis_index("i")
    rxq = [rxq0, rxq1]
    sbs, rxs = [sbs0, sbs1], [rxs0, rxs1]

    barrier = pltpu.get_barrier_semaphore()
    i = me & 3
    z = me >> 2
    for d in range(1, 4):
        pl.semaphore_signal(barrier, inc=1,
                            device_id=(_member(me, 0, (i + d) % 4),),
                            device_id_type=pl.DeviceIdType.MESH)
        pl.semaphore_signal(barrier, inc=1,
                            device_id=(_member(me, 1, (z + d) % 4),),
                            device_id_type=pl.DeviceIdType.MESH)
    pl.semaphore_wait(barrier, 6)

    qb = lax.broadcasted_iota(jnp.int32, (SQ, SQ), 0) // 64
    kb = lax.broadcasted_iota(jnp.int32, (SQ, SQ), 1) // 64
    bias = jnp.where(kb <= qb, 0.0, -1e9).astype(jnp.float32)

    sems = (rs0_s, rs0_r, rs1_s, rs1_r, ag0_s, ag0_r, ag1_s, ag1_r)
    ar0 = _BatchAllReduce(0, me, acc_ref, agb_ref, rxq, sbs, rxs, sems)
    ar1 = _BatchAllReduce(1, me, acc_ref, agb_ref, rxq, sbs, rxs, sems)

    _compute_batch(0, x_ref, wq_ref, k_ref, v_ref, wo_ref, acc_ref, bias)
    ar0.start_rs0()
    _compute_batch(1, x_ref, wq_ref, k_ref, v_ref, wo_ref, acc_ref, bias)

    ar0.finish_rs0(); ar0.start_rs1()
    ar1.start_rs0()
    ar0.finish_rs1(); ar0.start_ag0()
    ar1.finish_rs0(); ar1.start_rs1()
    ar0.finish_ag0(); ar0.start_ag1()
    ar1.finish_rs1(); ar1.start_ag0()
    ar0.finish_ag1()
    out_ref[0] = agb_ref[0:SQ, :].astype(jnp.float32)
    ar1.finish_ag0(); ar1.start_ag1()
    ar1.finish_ag1()
    out_ref[1] = agb_ref[SQ:ROWS, :].astype(jnp.float32)


def kernel(x, Wq, K_ext, V_ext, Wo):
    me = lax.axis_index("i")
    wq_loc = lax.dynamic_slice(Wq, (0, me * D_LOC), (Wq.shape[0], D_LOC))
    wo_loc = lax.dynamic_slice(Wo, (me * D_LOC, 0), (D_LOC, Wo.shape[1]))

    return pl.pallas_call(
        _body,
        out_shape=jax.ShapeDtypeStruct((B, SQ, D_MODEL), jnp.float32),
        in_specs=[pl.BlockSpec(memory_space=pltpu.VMEM)] * 5,
        out_specs=pl.BlockSpec(memory_space=pltpu.VMEM),
        scratch_shapes=[
            pltpu.VMEM((ROWS, D_MODEL), jnp.float32),
            pltpu.VMEM((ROWS, D_MODEL), jnp.bfloat16),
            pltpu.VMEM((B, 3, QR, COLS[0][1]), jnp.bfloat16),
            pltpu.VMEM((B, 3, QR, COLS[1][1]), jnp.bfloat16),
            pltpu.VMEM((B, 3, SR, COLS[0][1]), jnp.bfloat16),
            pltpu.VMEM((B, 3, SR, COLS[1][1]), jnp.bfloat16),
            pltpu.VMEM((B, 3, SR, COLS[0][1]), jnp.bfloat16),
            pltpu.VMEM((B, 3, SR, COLS[1][1]), jnp.bfloat16),
            pltpu.SemaphoreType.DMA((B, NSCHED, 3)),
            pltpu.SemaphoreType.DMA((B, NSCHED, 3)),
            pltpu.SemaphoreType.DMA((B, NSCHED, 3)),
            pltpu.SemaphoreType.DMA((B, NSCHED, 3)),
            pltpu.SemaphoreType.DMA((B, NSCHED, 3)),
            pltpu.SemaphoreType.DMA((B, NSCHED, 3)),
            pltpu.SemaphoreType.DMA((B, NSCHED, 3)),
            pltpu.SemaphoreType.DMA((B, NSCHED, 3)),
        ],
        compiler_params=pltpu.CompilerParams(collective_id=0),
    )(x, wq_loc, K_ext, V_ext, wo_loc)
